# baseline (speedup 1.0000x reference)
import functools

import jax
import jax.numpy as jnp
from jax import lax
from jax.experimental import pallas as pl
from jax.experimental.pallas import tpu as pltpu
from jax.experimental.pallas import tpu_sc as plsc

T = 32768
V = 32
NW = 32
ROWS = T // NW
HALO = 128
LROWS = ROWS + HALO

_DNUMS = lax.GatherDimensionNumbers(
    offset_dims=(), collapsed_slice_dims=(0,), start_index_map=(0,))


def _shuffle(vec, idx):
    return lax.gather(vec, idx[:, None], _DNUMS, slice_sizes=(1,),
                      mode=lax.GatherScatterMode.PROMISE_IN_BOUNDS)


def _sc_body(emt_hbm, out_hbm, emis_v, out_v, sem1, sem2):
    c = lax.axis_index("c")
    s = lax.axis_index("s")
    wid = s * 2 + c
    start = wid * ROWS

    off = jnp.where(wid > 0, HALO, 0)
    load_start = start - off
    cp1 = pltpu.async_copy(emt_hbm.at[:, pl.ds(load_start, 640)],
                           emis_v.at[:, pl.ds(0, 640)], sem1)
    cp2 = pltpu.async_copy(emt_hbm.at[:, pl.ds(load_start + 640, 512)],
                           emis_v.at[:, pl.ds(640, 512)], sem2)

    iota = lax.iota(jnp.int32, 16)
    shift_idx = jnp.maximum(iota - 1, 0)
    last_idx = iota * 0 + 15
    lane0 = iota == 0

    def argmax16(base):
        cur_max = emis_v[0, pl.ds(base, 16)]
        cur_id = jnp.zeros((16,), jnp.int32)
        for v in range(1, V):
            vals = emis_v[v, pl.ds(base, 16)]
            gt = vals > cur_max
            cur_max = jnp.where(gt, vals, cur_max)
            cur_id = jnp.where(gt, v, cur_id)
        return cur_id

    def collapse(cur_id, prev_ids):
        prev = jnp.where(lane0, _shuffle(prev_ids, last_idx),
                         _shuffle(cur_id, shift_idx))
        keep = (cur_id != prev) & (cur_id != 0)
        return jnp.where(keep, cur_id, -1)

    cp1.wait()
    seed = jnp.where(wid > 0, argmax16(jnp.maximum(off - 16, 0)), -1)

    def quad_body(i, prev_ids):
        base = off + i * 64
        ids = [argmax16(base + 16 * k) for k in range(4)]
        out_v[pl.ds(i * 64, 16)] = collapse(ids[0], prev_ids)
        for k in range(1, 4):
            out_v[pl.ds(i * 64 + 16 * k, 16)] = collapse(ids[k], ids[k - 1])
        return ids[3]

    mid = lax.fori_loop(0, 8, quad_body, seed)
    cp2.wait()
    lax.fori_loop(8, ROWS // 64, quad_body, mid)

    pltpu.sync_copy(out_v, out_hbm.at[pl.ds(start, ROWS)])


_ctc_sc = functools.partial(
    pl.kernel,
    out_type=jax.ShapeDtypeStruct((T,), jnp.int32),
    mesh=plsc.VectorSubcoreMesh(core_axis_name="c", subcore_axis_name="s"),
    compiler_params=pltpu.CompilerParams(
        use_tc_tiling_on_sc=True, needs_layout_passes=False),
    scratch_types=[
        pltpu.VMEM((V, LROWS), jnp.float32),
        pltpu.VMEM((ROWS,), jnp.int32),
        pltpu.SemaphoreType.DMA,
        pltpu.SemaphoreType.DMA,
    ],
)(_sc_body)


@jax.jit
def kernel(emission):
    return _ctc_sc(emission.T)


# speedup vs baseline: 1.0190x; 1.0190x over previous
import functools

import jax
import jax.numpy as jnp
from jax import lax
from jax.experimental import pallas as pl
from jax.experimental.pallas import tpu as pltpu
from jax.experimental.pallas import tpu_sc as plsc

T = 32768
V = 32
NW = 32
ROWS = T // NW
HALO = 128
LROWS = ROWS + HALO

_DNUMS = lax.GatherDimensionNumbers(
    offset_dims=(), collapsed_slice_dims=(0,), start_index_map=(0,))


def _shuffle(vec, idx):
    return lax.gather(vec, idx[:, None], _DNUMS, slice_sizes=(1,),
                      mode=lax.GatherScatterMode.PROMISE_IN_BOUNDS)


def _sc_body(emt_hbm, out_hbm, emis_v, out_v):
    c = lax.axis_index("c")
    s = lax.axis_index("s")
    wid = s * 2 + c
    start = wid * ROWS

    off = jnp.where(wid > 0, HALO, 0)
    load_start = start - off
    pltpu.sync_copy(emt_hbm.at[:, pl.ds(load_start, LROWS)], emis_v)

    iota = lax.iota(jnp.int32, 16)
    shift_idx = jnp.maximum(iota - 1, 0)
    last_idx = iota * 0 + 15
    lane0 = iota == 0

    def argmax16(base):
        cur_max = emis_v[0, pl.ds(base, 16)]
        cur_id = jnp.zeros((16,), jnp.int32)
        for v in range(1, V):
            vals = emis_v[v, pl.ds(base, 16)]
            gt = vals > cur_max
            cur_max = jnp.where(gt, vals, cur_max)
            cur_id = jnp.where(gt, v, cur_id)
        return cur_id

    def collapse(cur_id, prev_ids):
        prev = jnp.where(lane0, _shuffle(prev_ids, last_idx),
                         _shuffle(cur_id, shift_idx))
        keep = (cur_id != prev) & (cur_id != 0)
        return jnp.where(keep, cur_id, -1)

    seed = jnp.where(wid > 0, argmax16(jnp.maximum(off - 16, 0)), -1)

    def pair_body(i, prev_ids):
        base = off + i * 32
        ids_a = argmax16(base)
        ids_b = argmax16(base + 16)
        out_v[pl.ds(i * 32, 16)] = collapse(ids_a, prev_ids)
        out_v[pl.ds(i * 32 + 16, 16)] = collapse(ids_b, ids_a)
        return ids_b

    lax.fori_loop(0, ROWS // 32, pair_body, seed)

    pltpu.sync_copy(out_v, out_hbm.at[pl.ds(start, ROWS)])


_ctc_sc = functools.partial(
    pl.kernel,
    out_type=jax.ShapeDtypeStruct((T,), jnp.int32),
    mesh=plsc.VectorSubcoreMesh(core_axis_name="c", subcore_axis_name="s"),
    compiler_params=pltpu.CompilerParams(
        use_tc_tiling_on_sc=True, needs_layout_passes=False),
    scratch_types=[
        pltpu.VMEM((V, LROWS), jnp.float32),
        pltpu.VMEM((ROWS,), jnp.int32),
    ],
)(_sc_body)


@jax.jit
def kernel(emission):
    return _ctc_sc(emission.T)
